# baseline (speedup 1.0000x reference)
import functools

import jax
import jax.numpy as jnp
from jax import lax
from jax.experimental import pallas as pl
from jax.experimental.pallas import tpu as pltpu
from jax.experimental.pallas import tpu_sc as plsc

MAXLEN = 200
D = 64
LANES = 16
DG = D // LANES

_info = plsc.get_sparse_core_info()
NC = _info.num_cores
NS = _info.num_subcores
NW = NC * NS

B_ROWS_PER_CHUNK = 4
CHUNK = B_ROWS_PER_CHUNK * MAXLEN


def _make_kernel(n_flat):
    per_w = n_flat // NW
    n_chunks = per_w // CHUNK

    mesh = plsc.VectorSubcoreMesh(core_axis_name="c", subcore_axis_name="s")

    @functools.partial(
        pl.kernel,
        mesh=mesh,
        compiler_params=pltpu.CompilerParams(use_tc_tiling_on_sc=False),
        out_type=jax.ShapeDtypeStruct((n_flat, D), jnp.float32),
        scratch_types=[
            pltpu.VMEM((MAXLEN, D), jnp.float32),
            pltpu.VMEM((CHUNK,), jnp.int32),
            pltpu.VMEM((CHUNK, D), jnp.float32),
            pltpu.SemaphoreType.DMA,
        ],
    )
    def tok_pos_kernel(x_hbm, tok_hbm, pos_hbm, out_hbm, pos_v, idx_v, rows_v, sem):
        wid = lax.axis_index("s") * NC + lax.axis_index("c")
        base = wid * per_w

        pltpu.sync_copy(pos_hbm, pos_v)

        def chunk_body(i, carry):
            row0 = base + i * CHUNK
            pltpu.sync_copy(x_hbm.at[pl.ds(row0, CHUNK)], idx_v)
            pltpu.async_copy(tok_hbm.at[idx_v], rows_v, sem).wait()

            def add_body(t, c2):
                ps = [pos_v[t, pl.ds(c * LANES, LANES)] for c in range(DG)]
                for b in range(B_ROWS_PER_CHUNK):
                    r = b * MAXLEN + t
                    for c in range(DG):
                        sl = pl.ds(c * LANES, LANES)
                        rows_v[r, sl] = rows_v[r, sl] + ps[c]
                return c2

            lax.fori_loop(0, MAXLEN, add_body, 0)
            pltpu.sync_copy(rows_v, out_hbm.at[pl.ds(row0, CHUNK)])
            return carry

        lax.fori_loop(0, n_chunks, chunk_body, 0)

    return tok_pos_kernel


def kernel(x, token_table, pos_table):
    b, t = x.shape
    n_flat = b * t
    x_flat = x.reshape(n_flat).astype(jnp.int32)
    out = _make_kernel(n_flat)(x_flat, token_table, pos_table)
    return out.reshape(b, t, D)
